# baseline (speedup 1.0000x reference)
import functools

import jax
import jax.numpy as jnp
from jax import lax
from jax.experimental import pallas as pl
from jax.experimental.pallas import tpu as pltpu
from jax.experimental.pallas import tpu_sc as plsc

N, E, F, H, T = 10000, 320000, 128, 64, 64
NC, NS = 2, 16
NW = NC * NS
EPW = E // NW
CH = 100
HC = CH // 2
NCH = EPW // CH
E2 = E // 2
EBW = EPW // 2
STRIPE = N // NS
CW = 8

_mesh = plsc.VectorSubcoreMesh(core_axis_name="c", subcore_axis_name="s",
                               num_cores=NC, num_subcores=NS)
_sc_params = pltpu.CompilerParams(use_tc_tiling_on_sc=False)


def _rsqrt16(x):
    xh = x * 0.5
    i = lax.bitcast_convert_type(x, jnp.int32)
    i = jnp.int32(0x5F3759DF) - lax.shift_right_logical(i, 1)
    y = lax.bitcast_convert_type(i, jnp.float32)
    y = y * (1.5 - xh * y * y)
    y = y * (1.5 - xh * y * y)
    y = y * (1.5 - xh * y * y)
    return y


def _perm16(x, idx):
    return lax.gather(
        x, idx[:, None],
        lax.GatherDimensionNumbers(offset_dims=(), collapsed_slice_dims=(0,),
                                   start_index_map=(0,)),
        slice_sizes=(1,),
        mode=lax.GatherScatterMode.PROMISE_IN_BOUNDS)


def _splat_sum16(x):
    for s in (8, 4, 2, 1):
        idx = jnp.bitwise_xor(lax.iota(jnp.int32, 16), s)
        x = x + _perm16(x, idx)
    return x


@functools.partial(
    pl.kernel,
    out_type=(jax.ShapeDtypeStruct((NC, N, H), jnp.float32),
              jax.ShapeDtypeStruct((NC, N, CW), jnp.float32)),
    mesh=_mesh,
    compiler_params=_sc_params,
    scratch_types=[
        pltpu.VMEM((2, NCH, HC), jnp.int32),
        pltpu.VMEM((2, NCH, HC), jnp.int32),
        pltpu.VMEM((HC, 128), jnp.float32),
        pltpu.VMEM((HC, 128), jnp.float32),
        pltpu.VMEM((HC, H), jnp.float32),
        pltpu.VMEM((HC, H), jnp.float32),
        pltpu.VMEM((HC, H), jnp.float32),
        pltpu.VMEM((HC, H), jnp.float32),
        pltpu.VMEM((HC, H), jnp.float32),
        pltpu.VMEM((HC, H), jnp.float32),
        pltpu.VMEM((HC, CW), jnp.float32),
        pltpu.VMEM((2, H), jnp.float32),
        pltpu.VMEM_SHARED((N, H), jnp.float32),
        pltpu.VMEM_SHARED((N, CW), jnp.float32),
        pltpu.SemaphoreType.DMA,
        pltpu.SemaphoreType.DMA,
        pltpu.SemaphoreType.DMA,
        pltpu.SemaphoreType.DMA,
        pltpu.SemaphoreType.DMA,
        pltpu.SemaphoreType.DMA,
    ],
)
def _sc_fused(xa_hbm, src_hbm, dst_hbm, eb_hbm, gbe_hbm, zs_hbm, zc_hbm,
              ones_hbm, s_out, c_out,
              idx_s, idx_d, ebv0, ebv1, glo0, glo1, ghi0, ghi1, hlo, hhi,
              ones_v, gbe_v, s_sh, c_sh, se0, se1, slo0, slo1, shi0, shi1):
    c = lax.axis_index("c")
    s = lax.axis_index("s")
    wid = c * NS + s
    ebbase = wid * EBW

    pltpu.sync_copy(src_hbm.at[0, wid], idx_s.at[0])
    pltpu.sync_copy(src_hbm.at[1, wid], idx_s.at[1])
    pltpu.sync_copy(dst_hbm.at[0, wid], idx_d.at[0])
    pltpu.sync_copy(dst_hbm.at[1, wid], idx_d.at[1])
    pltpu.sync_copy(ones_hbm, ones_v)
    pltpu.sync_copy(gbe_hbm, gbe_v)
    pltpu.sync_copy(zs_hbm, s_sh.at[pl.ds(s * STRIPE, STRIPE)])
    pltpu.sync_copy(zc_hbm, c_sh.at[pl.ds(s * STRIPE, STRIPE)])
    plsc.subcore_barrier()

    gk = [gbe_v[0, pl.ds(16 * k, 16)] for k in range(4)]
    bek = [gbe_v[1, pl.ds(16 * k, 16)] for k in range(4)]

    def fire(j, ebv, glo, ghi, sem_e, sem_lo, sem_hi):
        pltpu.async_copy(eb_hbm.at[pl.ds(ebbase + j * HC, HC)], ebv, sem_e)
        pltpu.async_copy(xa_hbm.at[idx_s.at[0, j]], glo, sem_lo)
        pltpu.async_copy(xa_hbm.at[idx_s.at[1, j]], ghi, sem_hi)

    def wait(j, ebv, glo, ghi, sem_e, sem_lo, sem_hi):
        pltpu.make_async_copy(eb_hbm.at[pl.ds(ebbase, HC)], ebv, sem_e).wait()
        pltpu.make_async_copy(xa_hbm.at[idx_s.at[0, j]], glo, sem_lo).wait()
        pltpu.make_async_copy(xa_hbm.at[idx_s.at[1, j]], ghi, sem_hi).wait()

    def compute_and_scatter(j, ebv, glo, ghi):
        @pl.loop(0, HC)
        def _row(r):
            for half, gxv, hv in ((0, glo, hlo), (1, ghi, hhi)):
                ofs = half * H
                t = [gxv[r, pl.ds(16 * k, 16)]
                     + ebv[r, pl.ds(ofs + 16 * k, 16)] for k in range(4)]
                t = [jnp.maximum(tk, 0.01 * tk) for tk in t]
                mv = _splat_sum16(t[0] + t[1] + t[2] + t[3]) * (1.0 / H)
                qv = _splat_sum16(t[0] * t[0] + t[1] * t[1]
                                  + t[2] * t[2] + t[3] * t[3]) * (1.0 / H)
                rv = _rsqrt16(qv - mv * mv + 1e-5)
                for k in range(4):
                    hv[r, pl.ds(16 * k, 16)] = \
                        (t[k] - mv) * (rv * gk[k]) + bek[k]

        pltpu.sync_copy(hlo, s_sh.at[idx_d.at[0, j]], add=True)
        pltpu.sync_copy(hhi, s_sh.at[idx_d.at[1, j]], add=True)
        pltpu.sync_copy(ones_v, c_sh.at[idx_d.at[0, j]], add=True)
        pltpu.sync_copy(ones_v, c_sh.at[idx_d.at[1, j]], add=True)

    fire(0, ebv0, glo0, ghi0, se0, slo0, shi0)

    @pl.loop(0, NCH, step=2)
    def _chunk(j):
        fire(j + 1, ebv1, glo1, ghi1, se1, slo1, shi1)
        wait(j, ebv0, glo0, ghi0, se0, slo0, shi0)
        compute_and_scatter(j, ebv0, glo0, ghi0)

        @pl.when(j + 2 < NCH)
        def _():
            fire(j + 2, ebv0, glo0, ghi0, se0, slo0, shi0)

        wait(j + 1, ebv1, glo1, ghi1, se1, slo1, shi1)
        compute_and_scatter(j + 1, ebv1, glo1, ghi1)

    plsc.subcore_barrier()
    pltpu.sync_copy(s_sh.at[pl.ds(s * STRIPE, STRIPE)],
                    s_out.at[c, pl.ds(s * STRIPE, STRIPE)])
    pltpu.sync_copy(c_sh.at[pl.ds(s * STRIPE, STRIPE)],
                    c_out.at[c, pl.ds(s * STRIPE, STRIPE)])


def _xa_body(x_ref, w_ref, o_ref):
    o_ref[...] = jnp.dot(x_ref[...], w_ref[...],
                         preferred_element_type=jnp.float32)


def _eb_body(ea_ref, w_ref, b_ref, o_ref):
    lo = jnp.dot(ea_ref[0], w_ref[...],
                 preferred_element_type=jnp.float32) + b_ref[...]
    hi = jnp.dot(ea_ref[1], w_ref[...],
                 preferred_element_type=jnp.float32) + b_ref[...]
    o_ref[...] = jnp.concatenate([lo, hi], axis=1)


def _node_body(x_ref, sp_ref, cp_ref, w1b_ref, b1b_ref, w2x_ref, w2a_ref,
               b2a_ref, g2_ref, be2_ref, w2b_ref, b2b_ref, o_ref):
    ssum = sp_ref[0] + sp_ref[1]
    cnt = cp_ref[0, :, 0:1] + cp_ref[1, :, 0:1]
    hbar = ssum / jnp.maximum(cnt, 1.0)
    agg = jnp.dot(hbar, w1b_ref[...], preferred_element_type=jnp.float32) \
        + b1b_ref[...]
    agg = jnp.where(cnt > 0, agg, 0.0)
    t = jnp.dot(x_ref[...], w2x_ref[...], preferred_element_type=jnp.float32) \
        + jnp.dot(agg, w2a_ref[...], preferred_element_type=jnp.float32) \
        + b2a_ref[...]
    t = jnp.maximum(t, 0.01 * t)
    m = jnp.sum(t, axis=-1, keepdims=True) * (1.0 / H)
    v = jnp.sum(t * t, axis=-1, keepdims=True) * (1.0 / H) - m * m
    t = (t - m) * (lax.rsqrt(v + 1e-5) * g2_ref[...]) + be2_ref[...]
    o_ref[...] = jnp.dot(t, w2b_ref[...], preferred_element_type=jnp.float32) \
        + b2b_ref[...]


_BN = 2000
_BE = 4000


def _const_spec(shape):
    nd = len(shape)
    return pl.BlockSpec(shape, lambda *i: (0,) * nd)


def kernel(x, edge_idx, edge_attr, W1a, b1a, g1, be1, W1b, b1b,
           W2a, b2a, g2, be2, W2b, b2b):
    src = edge_idx[0].reshape(2, NW, NCH, HC)
    dst = edge_idx[1].reshape(2, NW, NCH, HC)
    w1x = W1a[:, :F].T
    w1e = W1a[:, F:].T
    gbe = jnp.stack([g1, be1])
    zs = jnp.zeros((STRIPE, H), jnp.float32)
    zc = jnp.zeros((STRIPE, CW), jnp.float32)
    ones = jnp.ones((HC, CW), jnp.float32)

    xa = pl.pallas_call(
        _xa_body,
        grid=(N // _BN,),
        in_specs=[pl.BlockSpec((_BN, F), lambda i: (i, 0)),
                  _const_spec((F, H))],
        out_specs=pl.BlockSpec((_BN, H), lambda i: (i, 0)),
        out_shape=jax.ShapeDtypeStruct((N, H), jnp.float32),
    )(x, w1x)

    eb = pl.pallas_call(
        _eb_body,
        grid=(E2 // _BE,),
        in_specs=[pl.BlockSpec((2, _BE, H), lambda i: (0, i, 0)),
                  _const_spec((H, H)), _const_spec((1, H))],
        out_specs=pl.BlockSpec((_BE, 128), lambda i: (i, 0)),
        out_shape=jax.ShapeDtypeStruct((E2, 128), jnp.float32),
    )(edge_attr.reshape(2, E2, H), w1e, b1a[None])

    s_parts, c_parts = _sc_fused(xa, src, dst, eb, gbe, zs, zc, ones)

    o = pl.pallas_call(
        _node_body,
        grid=(N // _BN,),
        in_specs=[pl.BlockSpec((_BN, F), lambda i: (i, 0)),
                  pl.BlockSpec((NC, _BN, H), lambda i: (0, i, 0)),
                  pl.BlockSpec((NC, _BN, CW), lambda i: (0, i, 0)),
                  _const_spec((H, H)), _const_spec((1, H)),
                  _const_spec((F, H)), _const_spec((H, H)),
                  _const_spec((1, H)), _const_spec((1, H)),
                  _const_spec((1, H)), _const_spec((H, T)),
                  _const_spec((1, T))],
        out_specs=pl.BlockSpec((_BN, T), lambda i: (i, 0)),
        out_shape=jax.ShapeDtypeStruct((N, T), jnp.float32),
    )(x, s_parts, c_parts, W1b.T, b1b[None], W2a[:, :F].T, W2a[:, F:].T,
      b2a[None], g2[None], be2[None], W2b.T, b2b[None])
    return o


# speedup vs baseline: 1.0956x; 1.0956x over previous
import functools

import jax
import jax.numpy as jnp
from jax import lax
from jax.experimental import pallas as pl
from jax.experimental.pallas import tpu as pltpu
from jax.experimental.pallas import tpu_sc as plsc

N, E, F, H, T = 10000, 320000, 128, 64, 64
NC, NS = 2, 16
NW = NC * NS
EPW = E // NW
CH = 200
HC = CH // 2
NCH = EPW // CH
E2 = E // 2
EBW = EPW // 2
STRIPE = N // NS
CW = 8

_mesh = plsc.VectorSubcoreMesh(core_axis_name="c", subcore_axis_name="s",
                               num_cores=NC, num_subcores=NS)
_sc_params = pltpu.CompilerParams(use_tc_tiling_on_sc=False)


def _rsqrt16(x):
    xh = x * 0.5
    i = lax.bitcast_convert_type(x, jnp.int32)
    i = jnp.int32(0x5F3759DF) - lax.shift_right_logical(i, 1)
    y = lax.bitcast_convert_type(i, jnp.float32)
    y = y * (1.5 - xh * y * y)
    y = y * (1.5 - xh * y * y)
    return y


def _perm16(x, idx):
    return lax.gather(
        x, idx[:, None],
        lax.GatherDimensionNumbers(offset_dims=(), collapsed_slice_dims=(0,),
                                   start_index_map=(0,)),
        slice_sizes=(1,),
        mode=lax.GatherScatterMode.PROMISE_IN_BOUNDS)


def _splat_sum16(x):
    for s in (8, 4, 2, 1):
        idx = jnp.bitwise_xor(lax.iota(jnp.int32, 16), s)
        x = x + _perm16(x, idx)
    return x


@functools.partial(
    pl.kernel,
    out_type=(jax.ShapeDtypeStruct((NC, N, H), jnp.float32),
              jax.ShapeDtypeStruct((NC, N, CW), jnp.float32)),
    mesh=_mesh,
    compiler_params=_sc_params,
    scratch_types=[
        pltpu.VMEM((2, NCH, HC), jnp.int32),
        pltpu.VMEM((2, NCH, HC), jnp.int32),
        pltpu.VMEM((HC, 128), jnp.float32),
        pltpu.VMEM((HC, 128), jnp.float32),
        pltpu.VMEM((HC, H), jnp.float32),
        pltpu.VMEM((HC, H), jnp.float32),
        pltpu.VMEM((HC, H), jnp.float32),
        pltpu.VMEM((HC, H), jnp.float32),
        pltpu.VMEM((HC, H), jnp.float32),
        pltpu.VMEM((HC, H), jnp.float32),
        pltpu.VMEM((HC, CW), jnp.float32),
        pltpu.VMEM((2, H), jnp.float32),
        pltpu.VMEM_SHARED((N, H), jnp.float32),
        pltpu.VMEM_SHARED((N, CW), jnp.float32),
        pltpu.SemaphoreType.DMA,
        pltpu.SemaphoreType.DMA,
        pltpu.SemaphoreType.DMA,
        pltpu.SemaphoreType.DMA,
        pltpu.SemaphoreType.DMA,
        pltpu.SemaphoreType.DMA,
        pltpu.SemaphoreType.DMA,
        pltpu.SemaphoreType.DMA,
        pltpu.SemaphoreType.DMA,
    ],
)
def _sc_fused(xa_hbm, src_hbm, dst_hbm, eb_hbm, gbe_hbm, zs_hbm, zc_hbm,
              ones_hbm, s_out, c_out,
              idx_s, idx_d, ebv0, ebv1, glo0, glo1, ghi0, ghi1, hlo, hhi,
              ones_v, gbe_v, s_sh, c_sh,
              se0, se1, slo0, slo1, shi0, shi1, wlo, whi, wcn):
    c = lax.axis_index("c")
    s = lax.axis_index("s")
    wid = c * NS + s
    ebbase = wid * EBW

    pltpu.sync_copy(src_hbm.at[0, wid], idx_s.at[0])
    pltpu.sync_copy(src_hbm.at[1, wid], idx_s.at[1])
    pltpu.sync_copy(dst_hbm.at[0, wid], idx_d.at[0])
    pltpu.sync_copy(dst_hbm.at[1, wid], idx_d.at[1])
    pltpu.sync_copy(ones_hbm, ones_v)
    pltpu.sync_copy(gbe_hbm, gbe_v)
    pltpu.sync_copy(zs_hbm, s_sh.at[pl.ds(s * STRIPE, STRIPE)])
    pltpu.sync_copy(zc_hbm, c_sh.at[pl.ds(s * STRIPE, STRIPE)])
    plsc.subcore_barrier()

    gk = [gbe_v[0, pl.ds(16 * k, 16)] for k in range(4)]
    bek = [gbe_v[1, pl.ds(16 * k, 16)] for k in range(4)]

    def fire(j, ebv, glo, ghi, sem_e, sem_lo, sem_hi):
        pltpu.async_copy(eb_hbm.at[pl.ds(ebbase + j * HC, HC)], ebv, sem_e)
        pltpu.async_copy(xa_hbm.at[idx_s.at[0, j]], glo, sem_lo)
        pltpu.async_copy(xa_hbm.at[idx_s.at[1, j]], ghi, sem_hi)

    def wait_in(j, ebv, glo, ghi, sem_e, sem_lo, sem_hi):
        pltpu.make_async_copy(eb_hbm.at[pl.ds(ebbase, HC)], ebv, sem_e).wait()
        pltpu.make_async_copy(xa_hbm.at[idx_s.at[0, j]], glo, sem_lo).wait()
        pltpu.make_async_copy(xa_hbm.at[idx_s.at[1, j]], ghi, sem_hi).wait()

    def wait_sc():
        pltpu.make_async_copy(hlo, s_sh.at[idx_d.at[0, 0]], wlo).wait()
        pltpu.make_async_copy(hhi, s_sh.at[idx_d.at[1, 0]], whi).wait()
        pltpu.make_async_copy(ones_v, c_sh.at[idx_d.at[0, 0]], wcn).wait()
        pltpu.make_async_copy(ones_v, c_sh.at[idx_d.at[1, 0]], wcn).wait()

    def compute(ebv, glo, ghi):
        @pl.loop(0, HC, unroll=2)
        def _row(r):
            for gxv, hv, ofs in ((glo, hlo, 0), (ghi, hhi, H)):
                t = [gxv[r, pl.ds(16 * k, 16)]
                     + ebv[r, pl.ds(ofs + 16 * k, 16)] for k in range(4)]
                t = [jnp.maximum(tk, 0.01 * tk) for tk in t]
                mv = _splat_sum16(t[0] + t[1] + t[2] + t[3]) * (1.0 / H)
                qv = _splat_sum16(t[0] * t[0] + t[1] * t[1]
                                  + t[2] * t[2] + t[3] * t[3]) * (1.0 / H)
                rv = _rsqrt16(qv - mv * mv + 1e-5)
                for k in range(4):
                    hv[r, pl.ds(16 * k, 16)] = \
                        (t[k] - mv) * (rv * gk[k]) + bek[k]

    def fire_sc(j):
        pltpu.async_copy(hlo, s_sh.at[idx_d.at[0, j]], wlo, add=True)
        pltpu.async_copy(hhi, s_sh.at[idx_d.at[1, j]], whi, add=True)
        pltpu.async_copy(ones_v, c_sh.at[idx_d.at[0, j]], wcn, add=True)
        pltpu.async_copy(ones_v, c_sh.at[idx_d.at[1, j]], wcn, add=True)

    fire(0, ebv0, glo0, ghi0, se0, slo0, shi0)

    @pl.loop(0, NCH, step=2)
    def _chunk(j):
        fire(j + 1, ebv1, glo1, ghi1, se1, slo1, shi1)
        wait_in(j, ebv0, glo0, ghi0, se0, slo0, shi0)

        @pl.when(j >= 2)
        def _():
            wait_sc()

        compute(ebv0, glo0, ghi0)
        fire_sc(j)

        @pl.when(j + 2 < NCH)
        def _():
            fire(j + 2, ebv0, glo0, ghi0, se0, slo0, shi0)

        wait_in(j + 1, ebv1, glo1, ghi1, se1, slo1, shi1)
        wait_sc()
        compute(ebv1, glo1, ghi1)
        fire_sc(j + 1)

    wait_sc()
    plsc.subcore_barrier()
    pltpu.sync_copy(s_sh.at[pl.ds(s * STRIPE, STRIPE)],
                    s_out.at[c, pl.ds(s * STRIPE, STRIPE)])
    pltpu.sync_copy(c_sh.at[pl.ds(s * STRIPE, STRIPE)],
                    c_out.at[c, pl.ds(s * STRIPE, STRIPE)])


def _xa_body(x_ref, w_ref, o_ref):
    o_ref[...] = jnp.dot(x_ref[...], w_ref[...],
                         preferred_element_type=jnp.float32)


def _eb_body(ea_ref, w_ref, b_ref, o_ref):
    lo = jnp.dot(ea_ref[0], w_ref[...],
                 preferred_element_type=jnp.float32) + b_ref[...]
    hi = jnp.dot(ea_ref[1], w_ref[...],
                 preferred_element_type=jnp.float32) + b_ref[...]
    o_ref[...] = jnp.concatenate([lo, hi], axis=1)


def _node_body(x_ref, sp_ref, cp_ref, w1b_ref, b1b_ref, w2x_ref, w2a_ref,
               b2a_ref, g2_ref, be2_ref, w2b_ref, b2b_ref, o_ref):
    ssum = sp_ref[0] + sp_ref[1]
    cnt = cp_ref[0, :, 0:1] + cp_ref[1, :, 0:1]
    hbar = ssum / jnp.maximum(cnt, 1.0)
    agg = jnp.dot(hbar, w1b_ref[...], preferred_element_type=jnp.float32) \
        + b1b_ref[...]
    agg = jnp.where(cnt > 0, agg, 0.0)
    t = jnp.dot(x_ref[...], w2x_ref[...], preferred_element_type=jnp.float32) \
        + jnp.dot(agg, w2a_ref[...], preferred_element_type=jnp.float32) \
        + b2a_ref[...]
    t = jnp.maximum(t, 0.01 * t)
    m = jnp.sum(t, axis=-1, keepdims=True) * (1.0 / H)
    v = jnp.sum(t * t, axis=-1, keepdims=True) * (1.0 / H) - m * m
    t = (t - m) * (lax.rsqrt(v + 1e-5) * g2_ref[...]) + be2_ref[...]
    o_ref[...] = jnp.dot(t, w2b_ref[...], preferred_element_type=jnp.float32) \
        + b2b_ref[...]


_BN = 2000
_BE = 4000


def _const_spec(shape):
    nd = len(shape)
    return pl.BlockSpec(shape, lambda *i: (0,) * nd)


def kernel(x, edge_idx, edge_attr, W1a, b1a, g1, be1, W1b, b1b,
           W2a, b2a, g2, be2, W2b, b2b):
    src = edge_idx[0].reshape(2, NW, NCH, HC)
    dst = edge_idx[1].reshape(2, NW, NCH, HC)
    w1x = W1a[:, :F].T
    w1e = W1a[:, F:].T
    gbe = jnp.stack([g1, be1])
    zs = jnp.zeros((STRIPE, H), jnp.float32)
    zc = jnp.zeros((STRIPE, CW), jnp.float32)
    ones = jnp.ones((HC, CW), jnp.float32)

    xa = pl.pallas_call(
        _xa_body,
        grid=(N // _BN,),
        in_specs=[pl.BlockSpec((_BN, F), lambda i: (i, 0)),
                  _const_spec((F, H))],
        out_specs=pl.BlockSpec((_BN, H), lambda i: (i, 0)),
        out_shape=jax.ShapeDtypeStruct((N, H), jnp.float32),
    )(x, w1x)

    eb = pl.pallas_call(
        _eb_body,
        grid=(E2 // _BE,),
        in_specs=[pl.BlockSpec((2, _BE, H), lambda i: (0, i, 0)),
                  _const_spec((H, H)), _const_spec((1, H))],
        out_specs=pl.BlockSpec((_BE, 128), lambda i: (i, 0)),
        out_shape=jax.ShapeDtypeStruct((E2, 128), jnp.float32),
    )(edge_attr.reshape(2, E2, H), w1e, b1a[None])

    s_parts, c_parts = _sc_fused(xa, src, dst, eb, gbe, zs, zc, ones)

    o = pl.pallas_call(
        _node_body,
        grid=(N // _BN,),
        in_specs=[pl.BlockSpec((_BN, F), lambda i: (i, 0)),
                  pl.BlockSpec((NC, _BN, H), lambda i: (0, i, 0)),
                  pl.BlockSpec((NC, _BN, CW), lambda i: (0, i, 0)),
                  _const_spec((H, H)), _const_spec((1, H)),
                  _const_spec((F, H)), _const_spec((H, H)),
                  _const_spec((1, H)), _const_spec((1, H)),
                  _const_spec((1, H)), _const_spec((H, T)),
                  _const_spec((1, T))],
        out_specs=pl.BlockSpec((_BN, T), lambda i: (i, 0)),
        out_shape=jax.ShapeDtypeStruct((N, T), jnp.float32),
    )(x, s_parts, c_parts, W1b.T, b1b[None], W2a[:, :F].T, W2a[:, F:].T,
      b2a[None], g2[None], be2[None], W2b.T, b2b[None])
    return o


# speedup vs baseline: 1.0960x; 1.0003x over previous
import functools

import jax
import jax.numpy as jnp
from jax import lax
from jax.experimental import pallas as pl
from jax.experimental.pallas import tpu as pltpu
from jax.experimental.pallas import tpu_sc as plsc

N, E, F, H, T = 10000, 320000, 128, 64, 64
NC, NS = 2, 16
NW = NC * NS
EPW = E // NW
CH = 200
HC = CH // 2
NCH = EPW // CH
E2 = E // 2
EBW = EPW // 2
STRIPE = N // NS
CW = 8

_mesh = plsc.VectorSubcoreMesh(core_axis_name="c", subcore_axis_name="s",
                               num_cores=NC, num_subcores=NS)
_sc_params = pltpu.CompilerParams(use_tc_tiling_on_sc=False)


def _rsqrt16(x):
    xh = x * 0.5
    i = lax.bitcast_convert_type(x, jnp.int32)
    i = jnp.int32(0x5F3759DF) - lax.shift_right_logical(i, 1)
    y = lax.bitcast_convert_type(i, jnp.float32)
    y = y * (1.5 - xh * y * y)
    y = y * (1.5 - xh * y * y)
    return y


def _perm16(x, idx):
    return lax.gather(
        x, idx[:, None],
        lax.GatherDimensionNumbers(offset_dims=(), collapsed_slice_dims=(0,),
                                   start_index_map=(0,)),
        slice_sizes=(1,),
        mode=lax.GatherScatterMode.PROMISE_IN_BOUNDS)


def _splat_sum16(x):
    for s in (8, 4, 2, 1):
        idx = jnp.bitwise_xor(lax.iota(jnp.int32, 16), s)
        x = x + _perm16(x, idx)
    return x


@functools.partial(
    pl.kernel,
    out_type=(jax.ShapeDtypeStruct((NC, N, H), jnp.float32),
              jax.ShapeDtypeStruct((NC, N, CW), jnp.float32)),
    mesh=_mesh,
    compiler_params=_sc_params,
    scratch_types=[
        pltpu.VMEM((2 * NCH, HC), jnp.int32),
        pltpu.VMEM((2 * NCH, HC), jnp.int32),
        pltpu.VMEM((HC, 128), jnp.float32),
        pltpu.VMEM((HC, 128), jnp.float32),
        pltpu.VMEM((HC, H), jnp.float32),
        pltpu.VMEM((HC, H), jnp.float32),
        pltpu.VMEM((HC, H), jnp.float32),
        pltpu.VMEM((HC, H), jnp.float32),
        pltpu.VMEM((HC, H), jnp.float32),
        pltpu.VMEM((HC, H), jnp.float32),
        pltpu.VMEM((HC, CW), jnp.float32),
        pltpu.VMEM((2, H), jnp.float32),
        pltpu.VMEM_SHARED((N, H), jnp.float32),
        pltpu.VMEM_SHARED((N, CW), jnp.float32),
        pltpu.SemaphoreType.DMA,
        pltpu.SemaphoreType.DMA,
        pltpu.SemaphoreType.DMA,
        pltpu.SemaphoreType.DMA,
        pltpu.SemaphoreType.DMA,
        pltpu.SemaphoreType.DMA,
        pltpu.SemaphoreType.DMA,
        pltpu.SemaphoreType.DMA,
        pltpu.SemaphoreType.DMA,
    ],
)
def _sc_fused(xa_hbm, src_hbm, dst_hbm, eb_hbm, gbe_hbm, zs_hbm, zc_hbm,
              ones_hbm, s_out, c_out,
              idx_s, idx_d, ebv0, ebv1, glo0, glo1, ghi0, ghi1, hlo, hhi,
              ones_v, gbe_v, s_sh, c_sh,
              se0, se1, slo0, slo1, shi0, shi1, wlo, whi, wcn):
    c = lax.axis_index("c")
    s = lax.axis_index("s")
    wid = c * NS + s
    ebbase = wid * EBW

    pltpu.sync_copy(src_hbm.at[0, wid], idx_s.at[pl.ds(0, NCH)])
    pltpu.sync_copy(src_hbm.at[1, wid], idx_s.at[pl.ds(NCH, NCH)])
    pltpu.sync_copy(dst_hbm.at[0, wid], idx_d.at[pl.ds(0, NCH)])
    pltpu.sync_copy(dst_hbm.at[1, wid], idx_d.at[pl.ds(NCH, NCH)])
    pltpu.sync_copy(ones_hbm, ones_v)
    pltpu.sync_copy(gbe_hbm, gbe_v)
    pltpu.sync_copy(zs_hbm, s_sh.at[pl.ds(s * STRIPE, STRIPE)])
    pltpu.sync_copy(zc_hbm, c_sh.at[pl.ds(s * STRIPE, STRIPE)])
    plsc.subcore_barrier()

    gk = [gbe_v[0, pl.ds(16 * k, 16)] for k in range(4)]
    bek = [gbe_v[1, pl.ds(16 * k, 16)] for k in range(4)]

    def fire(j, ebv, glo, ghi, sem_e, sem_lo, sem_hi):
        pltpu.async_copy(eb_hbm.at[pl.ds(ebbase + j * HC, HC)], ebv, sem_e)
        pltpu.async_copy(xa_hbm.at[idx_s.at[j]], glo, sem_lo)
        pltpu.async_copy(xa_hbm.at[idx_s.at[NCH + j]], ghi, sem_hi)

    def wait_in(j, ebv, glo, ghi, sem_e, sem_lo, sem_hi):
        pltpu.make_async_copy(eb_hbm.at[pl.ds(ebbase, HC)], ebv, sem_e).wait()
        pltpu.make_async_copy(xa_hbm.at[idx_s.at[j]], glo, sem_lo).wait()
        pltpu.make_async_copy(xa_hbm.at[idx_s.at[NCH + j]], ghi, sem_hi).wait()

    def wait_sc():
        pltpu.make_async_copy(hlo, s_sh.at[idx_d.at[0]], wlo).wait()
        pltpu.make_async_copy(hhi, s_sh.at[idx_d.at[0]], whi).wait()
        pltpu.make_async_copy(ones_v, c_sh.at[idx_d.at[0]], wcn).wait()
        pltpu.make_async_copy(ones_v, c_sh.at[idx_d.at[0]], wcn).wait()

    def compute(ebv, glo, ghi):
        @pl.loop(0, HC, unroll=2)
        def _row(r):
            for gxv, hv, ofs in ((glo, hlo, 0), (ghi, hhi, H)):
                t = [gxv[r, pl.ds(16 * k, 16)]
                     + ebv[r, pl.ds(ofs + 16 * k, 16)] for k in range(4)]
                t = [jnp.maximum(tk, 0.01 * tk) for tk in t]
                mv = _splat_sum16(t[0] + t[1] + t[2] + t[3]) * (1.0 / H)
                qv = _splat_sum16(t[0] * t[0] + t[1] * t[1]
                                  + t[2] * t[2] + t[3] * t[3]) * (1.0 / H)
                rv = _rsqrt16(qv - mv * mv + 1e-5)
                for k in range(4):
                    hv[r, pl.ds(16 * k, 16)] = \
                        (t[k] - mv) * (rv * gk[k]) + bek[k]

    def fire_sc(j):
        pltpu.async_copy(hlo, s_sh.at[idx_d.at[j]], wlo, add=True)
        pltpu.async_copy(hhi, s_sh.at[idx_d.at[NCH + j]], whi, add=True)
        pltpu.async_copy(ones_v, c_sh.at[idx_d.at[j]], wcn, add=True)
        pltpu.async_copy(ones_v, c_sh.at[idx_d.at[NCH + j]], wcn, add=True)

    fire(0, ebv0, glo0, ghi0, se0, slo0, shi0)

    @pl.loop(0, NCH, step=2)
    def _chunk(j):
        fire(j + 1, ebv1, glo1, ghi1, se1, slo1, shi1)
        wait_in(j, ebv0, glo0, ghi0, se0, slo0, shi0)

        @pl.when(j >= 2)
        def _():
            wait_sc()

        compute(ebv0, glo0, ghi0)
        fire_sc(j)

        @pl.when(j + 2 < NCH)
        def _():
            fire(j + 2, ebv0, glo0, ghi0, se0, slo0, shi0)

        wait_in(j + 1, ebv1, glo1, ghi1, se1, slo1, shi1)
        wait_sc()
        compute(ebv1, glo1, ghi1)
        fire_sc(j + 1)

    wait_sc()
    plsc.subcore_barrier()
    pltpu.sync_copy(s_sh.at[pl.ds(s * STRIPE, STRIPE)],
                    s_out.at[c, pl.ds(s * STRIPE, STRIPE)])
    pltpu.sync_copy(c_sh.at[pl.ds(s * STRIPE, STRIPE)],
                    c_out.at[c, pl.ds(s * STRIPE, STRIPE)])


def _xa_body(x_ref, w_ref, o_ref):
    o_ref[...] = jnp.dot(x_ref[...], w_ref[...],
                         preferred_element_type=jnp.float32)


def _eb_body(ea_ref, w_ref, b_ref, o_ref):
    lo = jnp.dot(ea_ref[0], w_ref[...],
                 preferred_element_type=jnp.float32) + b_ref[...]
    hi = jnp.dot(ea_ref[1], w_ref[...],
                 preferred_element_type=jnp.float32) + b_ref[...]
    o_ref[...] = jnp.concatenate([lo, hi], axis=1)


def _node_body(x_ref, sp_ref, cp_ref, w1b_ref, b1b_ref, w2x_ref, w2a_ref,
               b2a_ref, g2_ref, be2_ref, w2b_ref, b2b_ref, o_ref):
    ssum = sp_ref[0] + sp_ref[1]
    cnt = cp_ref[0, :, 0:1] + cp_ref[1, :, 0:1]
    hbar = ssum / jnp.maximum(cnt, 1.0)
    agg = jnp.dot(hbar, w1b_ref[...], preferred_element_type=jnp.float32) \
        + b1b_ref[...]
    agg = jnp.where(cnt > 0, agg, 0.0)
    t = jnp.dot(x_ref[...], w2x_ref[...], preferred_element_type=jnp.float32) \
        + jnp.dot(agg, w2a_ref[...], preferred_element_type=jnp.float32) \
        + b2a_ref[...]
    t = jnp.maximum(t, 0.01 * t)
    m = jnp.sum(t, axis=-1, keepdims=True) * (1.0 / H)
    v = jnp.sum(t * t, axis=-1, keepdims=True) * (1.0 / H) - m * m
    t = (t - m) * (lax.rsqrt(v + 1e-5) * g2_ref[...]) + be2_ref[...]
    o_ref[...] = jnp.dot(t, w2b_ref[...], preferred_element_type=jnp.float32) \
        + b2b_ref[...]


_BN = 2000
_BE = 4000


def _const_spec(shape):
    nd = len(shape)
    return pl.BlockSpec(shape, lambda *i: (0,) * nd)


def kernel(x, edge_idx, edge_attr, W1a, b1a, g1, be1, W1b, b1b,
           W2a, b2a, g2, be2, W2b, b2b):
    src = edge_idx[0].reshape(2, NW, NCH, HC)
    dst = edge_idx[1].reshape(2, NW, NCH, HC)
    w1x = W1a[:, :F].T
    w1e = W1a[:, F:].T
    gbe = jnp.stack([g1, be1])
    zs = jnp.zeros((STRIPE, H), jnp.float32)
    zc = jnp.zeros((STRIPE, CW), jnp.float32)
    ones = jnp.ones((HC, CW), jnp.float32)

    xa = pl.pallas_call(
        _xa_body,
        grid=(N // _BN,),
        in_specs=[pl.BlockSpec((_BN, F), lambda i: (i, 0)),
                  _const_spec((F, H))],
        out_specs=pl.BlockSpec((_BN, H), lambda i: (i, 0)),
        out_shape=jax.ShapeDtypeStruct((N, H), jnp.float32),
    )(x, w1x)

    eb = pl.pallas_call(
        _eb_body,
        grid=(E2 // _BE,),
        in_specs=[pl.BlockSpec((2, _BE, H), lambda i: (0, i, 0)),
                  _const_spec((H, H)), _const_spec((1, H))],
        out_specs=pl.BlockSpec((_BE, 128), lambda i: (i, 0)),
        out_shape=jax.ShapeDtypeStruct((E2, 128), jnp.float32),
    )(edge_attr.reshape(2, E2, H), w1e, b1a[None])

    s_parts, c_parts = _sc_fused(xa, src, dst, eb, gbe, zs, zc, ones)

    o = pl.pallas_call(
        _node_body,
        grid=(N // _BN,),
        in_specs=[pl.BlockSpec((_BN, F), lambda i: (i, 0)),
                  pl.BlockSpec((NC, _BN, H), lambda i: (0, i, 0)),
                  pl.BlockSpec((NC, _BN, CW), lambda i: (0, i, 0)),
                  _const_spec((H, H)), _const_spec((1, H)),
                  _const_spec((F, H)), _const_spec((H, H)),
                  _const_spec((1, H)), _const_spec((1, H)),
                  _const_spec((1, H)), _const_spec((H, T)),
                  _const_spec((1, T))],
        out_specs=pl.BlockSpec((_BN, T), lambda i: (i, 0)),
        out_shape=jax.ShapeDtypeStruct((N, T), jnp.float32),
    )(x, s_parts, c_parts, W1b.T, b1b[None], W2a[:, :F].T, W2a[:, F:].T,
      b2a[None], g2[None], be2[None], W2b.T, b2b[None])
    return o


# speedup vs baseline: 1.1576x; 1.0563x over previous
import functools

import jax
import jax.numpy as jnp
from jax import lax
from jax.experimental import pallas as pl
from jax.experimental.pallas import tpu as pltpu
from jax.experimental.pallas import tpu_sc as plsc

N, E, F, H, T = 10000, 320000, 128, 64, 64
NC, NS = 2, 16
NW = NC * NS
EPW = E // NW
CH = 100
NCH = EPW // CH
E2 = E // 2
EBW = EPW // 2
EBC = CH // 2
STRIPE = N // NS
CW = 8

_mesh = plsc.VectorSubcoreMesh(core_axis_name="c", subcore_axis_name="s",
                               num_cores=NC, num_subcores=NS)
_sc_params = pltpu.CompilerParams(use_tc_tiling_on_sc=False)


def _rsqrt16(x):
    xh = x * 0.5
    i = lax.bitcast_convert_type(x, jnp.int32)
    i = jnp.int32(0x5F3759DF) - lax.shift_right_logical(i, 1)
    y = lax.bitcast_convert_type(i, jnp.float32)
    y = y * (1.5 - xh * y * y)
    y = y * (1.5 - xh * y * y)
    y = y * (1.5 - xh * y * y)
    return y


def _perm16(x, idx):
    return lax.gather(
        x, idx[:, None],
        lax.GatherDimensionNumbers(offset_dims=(), collapsed_slice_dims=(0,),
                                   start_index_map=(0,)),
        slice_sizes=(1,),
        mode=lax.GatherScatterMode.PROMISE_IN_BOUNDS)


def _splat_sum16(x):
    for s in (8, 4, 2, 1):
        idx = jnp.bitwise_xor(lax.iota(jnp.int32, 16), s)
        x = x + _perm16(x, idx)
    return x


@functools.partial(
    pl.kernel,
    out_type=(jax.ShapeDtypeStruct((NC, N, H), jnp.float32),
              jax.ShapeDtypeStruct((NC, N, CW), jnp.float32)),
    mesh=_mesh,
    compiler_params=_sc_params,
    scratch_types=[
        pltpu.VMEM((NCH, CH), jnp.int32),
        pltpu.VMEM((NCH, CH), jnp.int32),
        pltpu.VMEM((EBC, 128), jnp.float32),
        pltpu.VMEM((EBC, 128), jnp.float32),
        pltpu.VMEM((CH, H), jnp.float32),
        pltpu.VMEM((CH, H), jnp.float32),
        pltpu.VMEM((CH, H), jnp.float32),
        pltpu.VMEM((CH, CW), jnp.float32),
        pltpu.VMEM((2, H), jnp.float32),
        pltpu.VMEM_SHARED((N, H), jnp.float32),
        pltpu.VMEM_SHARED((N, CW), jnp.float32),
        pltpu.SemaphoreType.DMA,
        pltpu.SemaphoreType.DMA,
        pltpu.SemaphoreType.DMA,
        pltpu.SemaphoreType.DMA,
    ],
)
def _sc_fused(xa_hbm, src_hbm, dst_hbm, eb_hbm, gbe_hbm, zs_hbm, zc_hbm,
              ones_hbm, s_out, c_out,
              idx_s, idx_d, ebv0, ebv1, gxv0, gxv1, hv, ones_v, gbe_v,
              s_sh, c_sh, se0, se1, sg0, sg1):
    c = lax.axis_index("c")
    s = lax.axis_index("s")
    wid = c * NS + s
    ebbase = wid * EBW

    pltpu.sync_copy(src_hbm.at[wid], idx_s)
    pltpu.sync_copy(dst_hbm.at[wid], idx_d)
    pltpu.sync_copy(ones_hbm, ones_v)
    pltpu.sync_copy(gbe_hbm, gbe_v)
    pltpu.sync_copy(zs_hbm, s_sh.at[pl.ds(s * STRIPE, STRIPE)])
    pltpu.sync_copy(zc_hbm, c_sh.at[pl.ds(s * STRIPE, STRIPE)])
    plsc.subcore_barrier()

    gk = [gbe_v[0, pl.ds(16 * k, 16)] for k in range(4)]
    bek = [gbe_v[1, pl.ds(16 * k, 16)] for k in range(4)]

    def fire(j, ebv, gxv, sem_e, sem_g):
        pltpu.async_copy(eb_hbm.at[pl.ds(ebbase + j * EBC, EBC)], ebv, sem_e)
        pltpu.async_copy(xa_hbm.at[idx_s.at[j]], gxv, sem_g)

    def wait(j, ebv, gxv, sem_e, sem_g):
        pltpu.make_async_copy(eb_hbm.at[pl.ds(ebbase, EBC)], ebv, sem_e).wait()
        pltpu.make_async_copy(xa_hbm.at[idx_s.at[j]], gxv, sem_g).wait()

    def compute_and_scatter(j, ebv, gxv):
        @pl.loop(0, EBC)
        def _row(r):
            for half in range(2):
                e = 2 * r + half
                ofs = half * H
                t = [gxv[e, pl.ds(16 * k, 16)]
                     + ebv[r, pl.ds(ofs + 16 * k, 16)] for k in range(4)]
                t = [jnp.maximum(tk, 0.01 * tk) for tk in t]
                mv = _splat_sum16(t[0] + t[1] + t[2] + t[3]) * (1.0 / H)
                qv = _splat_sum16(t[0] * t[0] + t[1] * t[1]
                                  + t[2] * t[2] + t[3] * t[3]) * (1.0 / H)
                rv = _rsqrt16(qv - mv * mv + 1e-5)
                for k in range(4):
                    hv[e, pl.ds(16 * k, 16)] = \
                        (t[k] - mv) * (rv * gk[k]) + bek[k]

        pltpu.sync_copy(hv, s_sh.at[idx_d.at[j]], add=True)
        pltpu.sync_copy(ones_v, c_sh.at[idx_d.at[j]], add=True)

    fire(0, ebv0, gxv0, se0, sg0)

    @pl.loop(0, NCH, step=2)
    def _chunk(j):
        fire(j + 1, ebv1, gxv1, se1, sg1)
        wait(j, ebv0, gxv0, se0, sg0)
        compute_and_scatter(j, ebv0, gxv0)

        @pl.when(j + 2 < NCH)
        def _():
            fire(j + 2, ebv0, gxv0, se0, sg0)

        wait(j + 1, ebv1, gxv1, se1, sg1)
        compute_and_scatter(j + 1, ebv1, gxv1)

    plsc.subcore_barrier()
    pltpu.sync_copy(s_sh.at[pl.ds(s * STRIPE, STRIPE)],
                    s_out.at[c, pl.ds(s * STRIPE, STRIPE)])
    pltpu.sync_copy(c_sh.at[pl.ds(s * STRIPE, STRIPE)],
                    c_out.at[c, pl.ds(s * STRIPE, STRIPE)])


def _xa_body(x_ref, w_ref, o_ref):
    o_ref[...] = jnp.dot(x_ref[...], w_ref[...],
                         preferred_element_type=jnp.float32)


def _eb_body(ea_lo_ref, ea_hi_ref, w_ref, b_ref, o_ref):
    lo = jnp.dot(ea_lo_ref[...], w_ref[...],
                 preferred_element_type=jnp.float32) + b_ref[...]
    hi = jnp.dot(ea_hi_ref[...], w_ref[...],
                 preferred_element_type=jnp.float32) + b_ref[...]
    o_ref[...] = jnp.concatenate([lo, hi], axis=1)


def _node_body(x_ref, sp_ref, cp_ref, w1b_ref, b1b_ref, w2x_ref, w2a_ref,
               b2a_ref, g2_ref, be2_ref, w2b_ref, b2b_ref, o_ref):
    ssum = sp_ref[0] + sp_ref[1]
    cnt = cp_ref[0, :, 0:1] + cp_ref[1, :, 0:1]
    hbar = ssum / jnp.maximum(cnt, 1.0)
    agg = jnp.dot(hbar, w1b_ref[...], preferred_element_type=jnp.float32) \
        + b1b_ref[...]
    agg = jnp.where(cnt > 0, agg, 0.0)
    t = jnp.dot(x_ref[...], w2x_ref[...], preferred_element_type=jnp.float32) \
        + jnp.dot(agg, w2a_ref[...], preferred_element_type=jnp.float32) \
        + b2a_ref[...]
    t = jnp.maximum(t, 0.01 * t)
    m = jnp.sum(t, axis=-1, keepdims=True) * (1.0 / H)
    v = jnp.sum(t * t, axis=-1, keepdims=True) * (1.0 / H) - m * m
    t = (t - m) * (lax.rsqrt(v + 1e-5) * g2_ref[...]) + be2_ref[...]
    o_ref[...] = jnp.dot(t, w2b_ref[...], preferred_element_type=jnp.float32) \
        + b2b_ref[...]


_BN = 2000
_BE = 4000


def _const_spec(shape):
    nd = len(shape)
    return pl.BlockSpec(shape, lambda i: (0,) * nd)


def kernel(x, edge_idx, edge_attr, W1a, b1a, g1, be1, W1b, b1b,
           W2a, b2a, g2, be2, W2b, b2b):
    src = jnp.stack([edge_idx[0, :E2], edge_idx[0, E2:]], axis=1) \
        .reshape(NW, NCH, CH)
    dst = jnp.stack([edge_idx[1, :E2], edge_idx[1, E2:]], axis=1) \
        .reshape(NW, NCH, CH)
    w1x = W1a[:, :F].T
    w1e = W1a[:, F:].T
    gbe = jnp.stack([g1, be1])
    zs = jnp.zeros((STRIPE, H), jnp.float32)
    zc = jnp.zeros((STRIPE, CW), jnp.float32)
    ones = jnp.ones((CH, CW), jnp.float32)

    xa = pl.pallas_call(
        _xa_body,
        grid=(N // _BN,),
        in_specs=[pl.BlockSpec((_BN, F), lambda i: (i, 0)),
                  _const_spec((F, H))],
        out_specs=pl.BlockSpec((_BN, H), lambda i: (i, 0)),
        out_shape=jax.ShapeDtypeStruct((N, H), jnp.float32),
    )(x, w1x)

    nblk = E2 // _BE
    eb = pl.pallas_call(
        _eb_body,
        grid=(nblk,),
        in_specs=[pl.BlockSpec((_BE, H), lambda i: (i, 0)),
                  pl.BlockSpec((_BE, H), lambda i: (i + nblk, 0)),
                  _const_spec((H, H)), _const_spec((1, H))],
        out_specs=pl.BlockSpec((_BE, 128), lambda i: (i, 0)),
        out_shape=jax.ShapeDtypeStruct((E2, 128), jnp.float32),
    )(edge_attr, edge_attr, w1e, b1a[None])

    s_parts, c_parts = _sc_fused(xa, src, dst, eb, gbe, zs, zc, ones)

    o = pl.pallas_call(
        _node_body,
        grid=(N // _BN,),
        in_specs=[pl.BlockSpec((_BN, F), lambda i: (i, 0)),
                  pl.BlockSpec((NC, _BN, H), lambda i: (0, i, 0)),
                  pl.BlockSpec((NC, _BN, CW), lambda i: (0, i, 0)),
                  _const_spec((H, H)), _const_spec((1, H)),
                  _const_spec((F, H)), _const_spec((H, H)),
                  _const_spec((1, H)), _const_spec((1, H)),
                  _const_spec((1, H)), _const_spec((H, T)),
                  _const_spec((1, T))],
        out_specs=pl.BlockSpec((_BN, T), lambda i: (i, 0)),
        out_shape=jax.ShapeDtypeStruct((N, T), jnp.float32),
    )(x, s_parts, c_parts, W1b.T, b1b[None], W2a[:, :F].T, W2a[:, F:].T,
      b2a[None], g2[None], be2[None], W2b.T, b2b[None])
    return o
